# Initial kernel scaffold; baseline (speedup 1.0000x reference)
import jax
import jax.numpy as jnp
from jax.experimental import pallas as pl


def kernel(u, s, v, grad_weight, I_U, I_V):
    raise NotImplementedError("write your pallas kernel here")


# speedup vs baseline: 2.8722x; 2.8722x over previous
import functools

import jax
import jax.numpy as jnp
from jax import lax
from jax.experimental import pallas as pl
from jax.experimental.pallas import tpu as pltpu

RANK = 128
K = 512


def _body(s_row_ref, s_col_ref, u_ref, v_ref, gw_ref, iu_ref, iv_ref, o_ref):
    a_row = jnp.abs(s_row_ref[0])
    a_col = jnp.abs(s_col_ref[0])
    i0 = lax.broadcasted_iota(jnp.int32, (K, K), 0)
    i1 = lax.broadcasted_iota(jnp.int32, (K, K), 1)
    eq = a_col == a_row
    beats_lane = (a_col > a_row) | (eq & (i0 < i1))
    rank_row = jnp.sum(beats_lane.astype(jnp.int32), axis=0, keepdims=True)
    sel_row = rank_row < RANK
    beats_sub = (a_row > a_col) | (eq & (i1 < i0))
    rank_col = jnp.sum(beats_sub.astype(jnp.int32), axis=1, keepdims=True)
    sel_col = rank_col < RANK
    pos_col = jnp.sum(((i1 < i0) & sel_row).astype(jnp.int32), axis=1,
                      keepdims=True)
    r_iota = lax.broadcasted_iota(jnp.int32, (K, RANK), 1)
    P = jnp.where(sel_col & (pos_col == r_iota), 1.0, 0.0).astype(jnp.float32)

    dot = functools.partial(lax.dot_general, preferred_element_type=jnp.float32)
    u = u_ref[0]
    v = v_ref[0]
    gw = gw_ref[0]
    iu = iu_ref[0]
    iv = iv_ref[0]
    ivp = dot(iv, P, (((1,), (0,)), ((), ())))
    u2 = dot(u, ivp, (((1,), (0,)), ((), ())))
    A = dot(u2, gw, (((0,), (0,)), ((), ())))
    iup = dot(P, iu, (((0,), (0,)), ((), ())))
    v2 = dot(iup, v, (((1,), (0,)), ((), ())))
    gs = jnp.sum(A * v2, axis=1, keepdims=True)
    o_ref[0] = dot(P, gs, (((1,), (0,)), ((), ())))


def kernel(u, s, v, grad_weight, I_U, I_V):
    p, q, k = s.shape
    b = p * q
    m, n = u.shape[2], v.shape[3]
    s_row = s.reshape(b, 1, k)
    s_col = s.reshape(b, k, 1)
    big = lambda x: x.reshape(b, x.shape[2], x.shape[3])
    mat_spec = pl.BlockSpec((1, m, k), lambda i: (i, 0, 0))
    out = pl.pallas_call(
        _body,
        grid=(b,),
        in_specs=[
            pl.BlockSpec((1, 1, k), lambda i: (i, 0, 0)),
            pl.BlockSpec((1, k, 1), lambda i: (i, 0, 0)),
            mat_spec,
            mat_spec,
            mat_spec,
            mat_spec,
            mat_spec,
        ],
        out_specs=pl.BlockSpec((1, k, 1), lambda i: (i, 0, 0)),
        out_shape=jax.ShapeDtypeStruct((b, k, 1), jnp.float32),
    )(s_row, s_col, big(u), big(v), big(grad_weight), big(I_U), big(I_V))
    return out.reshape(p, q, k)


# speedup vs baseline: 2.8839x; 1.0041x over previous
import functools

import jax
import jax.numpy as jnp
from jax import lax
from jax.experimental import pallas as pl
from jax.experimental.pallas import tpu as pltpu

RANK = 128
K = 512


def _body(s_row_ref, s_col_ref, u_ref, v_ref, gw_ref, iu_ref, iv_ref, o_ref):
    a_row = jnp.abs(s_row_ref[0])
    a_col = jnp.abs(s_col_ref[0])
    i0 = lax.broadcasted_iota(jnp.int32, (K, K), 0)
    i1 = lax.broadcasted_iota(jnp.int32, (K, K), 1)
    eq = a_col == a_row
    beats_lane = (a_col > a_row) | (eq & (i0 < i1))
    rank_row = jnp.sum(beats_lane.astype(jnp.int32), axis=0, keepdims=True)
    sel_row = rank_row < RANK
    beats_sub = (a_row > a_col) | (eq & (i1 < i0))
    rank_col = jnp.sum(beats_sub.astype(jnp.int32), axis=1, keepdims=True)
    sel_col = rank_col < RANK
    pos_col = jnp.sum(((i1 < i0) & sel_row).astype(jnp.int32), axis=1,
                      keepdims=True)
    r_iota = lax.broadcasted_iota(jnp.int32, (K, RANK), 1)
    P = jnp.where(sel_col & (pos_col == r_iota), 1.0, 0.0).astype(jnp.float32)

    dot = functools.partial(lax.dot_general, preferred_element_type=jnp.float32)
    bf = lambda x: x.astype(jnp.bfloat16)
    Pb = bf(P)
    u = bf(u_ref[0])
    v = bf(v_ref[0])
    gw = bf(gw_ref[0])
    iu = bf(iu_ref[0])
    iv = bf(iv_ref[0])
    ivp = dot(iv, Pb, (((1,), (0,)), ((), ())))
    u2 = dot(u, bf(ivp), (((1,), (0,)), ((), ())))
    A = dot(bf(u2), gw, (((0,), (0,)), ((), ())))
    iup = dot(Pb, iu, (((0,), (0,)), ((), ())))
    v2 = dot(bf(iup), v, (((1,), (0,)), ((), ())))
    gs = jnp.sum(A * v2, axis=1, keepdims=True)
    o_ref[0] = dot(P, gs, (((1,), (0,)), ((), ())))


def kernel(u, s, v, grad_weight, I_U, I_V):
    p, q, k = s.shape
    b = p * q
    m, n = u.shape[2], v.shape[3]
    s_row = s.reshape(b, 1, k)
    s_col = s.reshape(b, k, 1)
    big = lambda x: x.reshape(b, x.shape[2], x.shape[3])
    mat_spec = pl.BlockSpec((1, m, k), lambda i: (i, 0, 0))
    out = pl.pallas_call(
        _body,
        grid=(b,),
        in_specs=[
            pl.BlockSpec((1, 1, k), lambda i: (i, 0, 0)),
            pl.BlockSpec((1, k, 1), lambda i: (i, 0, 0)),
            mat_spec,
            mat_spec,
            mat_spec,
            mat_spec,
            mat_spec,
        ],
        out_specs=pl.BlockSpec((1, k, 1), lambda i: (i, 0, 0)),
        out_shape=jax.ShapeDtypeStruct((b, k, 1), jnp.float32),
    )(s_row, s_col, big(u), big(v), big(grad_weight), big(I_U), big(I_V))
    return out.reshape(p, q, k)
